# baseline (speedup 1.0000x reference)
import functools

import jax
import jax.numpy as jnp
from jax import lax
from jax.experimental import pallas as pl
from jax.experimental.pallas import tpu as pltpu
from jax.experimental.pallas import tpu_sc as plsc

V_NUM = 1000000
H_DIM = 32
BATCH = 16384
FIELDS = 26

_N = BATCH * FIELDS
_NW = 32
_PER_W = _N // _NW
_CH = 832
_NCH = _PER_W // _CH
_NBUF = 4


def _make_kernel():
    mesh = plsc.VectorSubcoreMesh(core_axis_name="c", subcore_axis_name="s")

    @functools.partial(
        pl.kernel,
        mesh=mesh,
        out_type=jax.ShapeDtypeStruct((_N, H_DIM), jnp.float32),
        compiler_params=pltpu.CompilerParams(use_tc_tiling_on_sc=False),
        scratch_types=(
            [pltpu.VMEM((_PER_W,), jnp.int32)]
            + [pltpu.VMEM((_CH, H_DIM), jnp.float32)] * _NBUF
            + [pltpu.SemaphoreType.DMA] * (2 * _NBUF)
        ),
    )
    def gather_kernel(idx_hbm, table_hbm, out_hbm, idx_v, *bufs):
        rows = bufs[:_NBUF]
        gsem = bufs[_NBUF:2 * _NBUF]
        osem = bufs[2 * _NBUF:]

        wid = lax.axis_index("s") * 2 + lax.axis_index("c")
        base = wid * _PER_W
        pltpu.sync_copy(idx_hbm.at[pl.ds(base, _PER_W)], idx_v)

        def g_start(j, b):
            pltpu.async_copy(
                table_hbm.at[idx_v.at[pl.ds(j * _CH, _CH)]], rows[b], gsem[b])

        for b in range(_NBUF):
            g_start(b, b)

        for j in range(_NCH):
            b = j % _NBUF
            pltpu.make_async_copy(
                table_hbm.at[idx_v.at[pl.ds(j * _CH, _CH)]], rows[b],
                gsem[b]).wait()
            out_slice = out_hbm.at[pl.ds(base + j * _CH, _CH)]
            pltpu.async_copy(rows[b], out_slice, osem[b])
            nxt = j + _NBUF
            if nxt < _NCH:
                pltpu.make_async_copy(rows[b], out_slice, osem[b]).wait()
                g_start(nxt, b)

        for j in range(max(_NCH - _NBUF, 0), _NCH):
            b = j % _NBUF
            pltpu.make_async_copy(
                rows[b], out_hbm.at[pl.ds(base + j * _CH, _CH)],
                osem[b]).wait()

    return gather_kernel


_KERNEL = _make_kernel()


@jax.jit
def kernel(x, table):
    idx = x.reshape(-1).astype(jnp.int32)
    out = _KERNEL(idx, table)
    return out.reshape(BATCH, FIELDS, H_DIM)


# speedup vs baseline: 1.2920x; 1.2920x over previous
import functools

import jax
import jax.numpy as jnp
from jax import lax
from jax.experimental import pallas as pl
from jax.experimental.pallas import tpu as pltpu
from jax.experimental.pallas import tpu_sc as plsc

V_NUM = 1000000
H_DIM = 32
BATCH = 16384
FIELDS = 26

_N = BATCH * FIELDS
_NW = 32
_PER_W = _N // _NW
_GRP = 128
_G_PER_W = _PER_W // _GRP
_GPC = 8
_CH = _GPC * _GRP
_NCH = _G_PER_W // _GPC
_NBUF = 3
_TCG = BATCH // _GRP


def _make_kernel():
    mesh = plsc.VectorSubcoreMesh(core_axis_name="c", subcore_axis_name="s")

    @functools.partial(
        pl.kernel,
        mesh=mesh,
        out_type=jax.ShapeDtypeStruct((FIELDS, H_DIM // 8, _TCG, 8, _GRP),
                                      jnp.float32),
        compiler_params=pltpu.CompilerParams(
            use_tc_tiling_on_sc=False, needs_layout_passes=False),
        scratch_types=(
            pltpu.VMEM((_PER_W,), jnp.int32),
            pltpu.VMEM((_NBUF * _CH, H_DIM), jnp.float32),
            pltpu.VMEM((2, H_DIM, 129), jnp.float32),
            pltpu.SemaphoreType.DMA((_NBUF,)),
            pltpu.SemaphoreType.DMA((2,)),
        ),
    )
    def gather_kernel(idx_hbm, table_hbm, out_hbm, idx_v, rows, tbuf,
                      gsem, wsem):
        wid = lax.axis_index("s") * 2 + lax.axis_index("c")
        base = wid * _PER_W
        g_base = wid * _G_PER_W
        pltpu.sync_copy(idx_hbm.at[pl.ds(base, _PER_W)], idx_v)

        def g_start(ch, slot):
            pltpu.async_copy(
                table_hbm.at[idx_v.at[pl.ds(ch * _CH, _CH)]],
                rows.at[pl.ds(slot * _CH, _CH)],
                gsem.at[slot])

        def g_wait(ch, slot):
            pltpu.make_async_copy(
                table_hbm.at[idx_v.at[pl.ds(ch * _CH, _CH)]],
                rows.at[pl.ds(slot * _CH, _CH)],
                gsem.at[slot]).wait()

        def t_writes(gg, p, do_wait):
            f = gg // _TCG
            tc = gg % _TCG
            for tr in range(H_DIM // 8):
                cpy = pltpu.make_async_copy(
                    tbuf.at[p, pl.ds(8 * tr, 8), pl.ds(0, _GRP)],
                    out_hbm.at[f, tr, tc],
                    wsem.at[p])
                if do_wait:
                    cpy.wait()
                else:
                    cpy.start()

        iota = lax.iota(jnp.int32, 16)

        for slot in range(_NBUF):
            g_start(slot, slot)

        def chunk_body(ch, _):
            slot = lax.rem(ch, _NBUF)
            g_wait(ch, slot)

            def group_body(g8, _):
                gl = ch * _GPC + g8
                gg = g_base + gl
                p = lax.rem(gl, 2)

                @pl.when(gl >= 2)
                def _():
                    t_writes(gg - 2, p, True)

                row0 = slot * _CH + g8 * _GRP
                for l in range(_GRP):
                    for k in range(H_DIM // 16):
                        vec = rows[row0 + l, pl.ds(16 * k, 16)]
                        plsc.store_scatter(
                            tbuf.at[p],
                            [16 * k + iota, jnp.full((16,), l, jnp.int32)],
                            vec)
                t_writes(gg, p, False)
                return _

            lax.fori_loop(0, _GPC, group_body, None, unroll=False)

            @pl.when(ch + _NBUF < _NCH)
            def _():
                g_start(ch + _NBUF, slot)
            return _

        lax.fori_loop(0, _NCH, chunk_body, None, unroll=False)

        for tail in range(2):
            gl = _G_PER_W - 2 + tail
            t_writes(g_base + gl, gl % 2, True)

    return gather_kernel


_KERNEL = _make_kernel()


@jax.jit
def kernel(x, table):
    idx = x.T.reshape(-1).astype(jnp.int32)
    out5 = _KERNEL(idx, table)
    return out5.transpose(2, 4, 0, 1, 3).reshape(BATCH, FIELDS, H_DIM)
